# Initial kernel scaffold; baseline (speedup 1.0000x reference)
import jax
import jax.numpy as jnp
from jax.experimental import pallas as pl


def kernel(feat, segment_ids, num_graphs):
    raise NotImplementedError("write your pallas kernel here")


# speedup vs baseline: 4.6679x; 4.6679x over previous
import functools

import jax
import jax.numpy as jnp
from jax import lax
from jax.experimental import pallas as pl
from jax.experimental.pallas import tpu as pltpu
from jax.experimental.pallas import tpu_sc as plsc

N_ROWS = 100000
D = 128
S = 256
NC = 2
NS = 16
NW = NC * NS
CH = 128
FULL = (N_ROWS // (NW * CH)) * NW * CH
CHUNKS = FULL // (NW * CH)
EXTRA = (N_ROWS - FULL) // CH
TAIL = N_ROWS - FULL - EXTRA * CH
TAIL_OFF = FULL + EXTRA * CH


def _sc_body(feat_hbm, seg_hbm, psums_hbm, pcnts_hbm,
             acc, cnt, rows, idx, idx32, ones, zbuf):
  c = lax.axis_index("c")
  s = lax.axis_index("s")
  w = s * NC + c

  zv = jnp.zeros((16,), jnp.float32)
  ov = jnp.ones((16,), jnp.float32)
  for i in range(16):
    for k in range(D // 16):
      zbuf[i, pl.ds(k * 16, 16)] = zv
  for i in range(CH):
    for k in range(D // 16):
      ones[i, pl.ds(k * 16, 16)] = ov

  sl = pl.ds(s * 16, 16)
  pltpu.sync_copy(zbuf, acc.at[sl])
  pltpu.sync_copy(zbuf, cnt.at[sl])
  plsc.subcore_barrier()

  def do_chunk(off):
    pltpu.sync_copy(seg_hbm.at[pl.ds(off, CH)], idx)
    pltpu.sync_copy(feat_hbm.at[pl.ds(off, CH)], rows)
    pltpu.sync_copy(rows, acc.at[idx], add=True)
    pltpu.sync_copy(ones, cnt.at[idx], add=True)

  def body(j, carry):
    do_chunk(w * (CHUNKS * CH) + j * CH)
    return carry

  lax.fori_loop(0, CHUNKS, body, 0)

  @pl.when(w < EXTRA)
  def _():
    do_chunk(FULL + w * CH)

  @pl.when(w == EXTRA)
  def _():
    pltpu.sync_copy(seg_hbm.at[pl.ds(TAIL_OFF, TAIL)], idx32)
    pltpu.sync_copy(feat_hbm.at[pl.ds(TAIL_OFF, TAIL)], rows.at[pl.ds(0, TAIL)])
    pltpu.sync_copy(rows.at[pl.ds(0, TAIL)], acc.at[idx32], add=True)
    pltpu.sync_copy(ones.at[pl.ds(0, TAIL)], cnt.at[idx32], add=True)

  plsc.subcore_barrier()

  pltpu.sync_copy(acc.at[sl], psums_hbm.at[c, sl])
  pltpu.sync_copy(cnt.at[sl], pcnts_hbm.at[c, sl])


_sc_pool = functools.partial(
    pl.kernel,
    out_type=(
        jax.ShapeDtypeStruct((NC, S, D), jnp.float32),
        jax.ShapeDtypeStruct((NC, S, D), jnp.float32),
    ),
    mesh=plsc.VectorSubcoreMesh(
        core_axis_name="c", subcore_axis_name="s",
        num_cores=NC, num_subcores=NS),
    scratch_types=[
        pltpu.VMEM_SHARED((S, D), jnp.float32),
        pltpu.VMEM_SHARED((S, D), jnp.float32),
        pltpu.VMEM((CH, D), jnp.float32),
        pltpu.VMEM((CH,), jnp.int32),
        pltpu.VMEM((TAIL,), jnp.int32),
        pltpu.VMEM((CH, D), jnp.float32),
        pltpu.VMEM((16, D), jnp.float32),
    ],
)(_sc_body)


def _combine_body(ps_ref, pc_ref, o_ref):
  sums = ps_ref[0] + ps_ref[1]
  counts = pc_ref[0, :, 0:1] + pc_ref[1, :, 0:1]
  o_ref[...] = sums / jnp.clip(counts, 1.0, None)


def kernel(feat, segment_ids, num_graphs):
  seg = segment_ids.astype(jnp.int32)
  psums, pcnts = _sc_pool(feat, seg)
  out = pl.pallas_call(
      _combine_body,
      out_shape=jax.ShapeDtypeStruct((S, D), jnp.float32),
  )(psums, pcnts)
  return out


# speedup vs baseline: 5.5913x; 1.1978x over previous
import functools

import jax
import jax.numpy as jnp
from jax import lax
from jax.experimental import pallas as pl
from jax.experimental.pallas import tpu as pltpu
from jax.experimental.pallas import tpu_sc as plsc

N_ROWS = 100000
D = 128
S = 256
NC = 2
NS = 16
NW = NC * NS
CH = 128
FULL = (N_ROWS // (NW * CH)) * NW * CH
CHUNKS = FULL // (NW * CH)
EXTRA = (N_ROWS - FULL) // CH
TAIL = N_ROWS - FULL - EXTRA * CH
TAIL_OFF = FULL + EXTRA * CH


def _sc_body(feat_hbm, seg_hbm, psums_hbm, pcnts_hbm,
             acc, rows, idx, idx32, cnt, zbuf):
  c = lax.axis_index("c")
  s = lax.axis_index("s")
  w = s * NC + c

  zv = jnp.zeros((16,), jnp.float32)
  ov = jnp.ones((16,), jnp.float32)
  for i in range(16):
    for k in range(D // 16):
      zbuf[i, pl.ds(k * 16, 16)] = zv
  for i in range(S // 16):
    cnt[pl.ds(i * 16, 16)] = zv

  sl = pl.ds(s * 16, 16)
  pltpu.sync_copy(zbuf, acc.at[sl])
  plsc.subcore_barrier()

  def count_ids(idx_ref, n):
    for k in range(n // 16):
      plsc.addupdate_scatter(cnt, [idx_ref[pl.ds(k * 16, 16)]], ov)

  def do_chunk(off):
    pltpu.sync_copy(seg_hbm.at[pl.ds(off, CH)], idx)
    pltpu.sync_copy(feat_hbm.at[pl.ds(off, CH)], rows)
    pltpu.sync_copy(rows, acc.at[idx], add=True)
    count_ids(idx, CH)

  def body(j, carry):
    do_chunk(w * (CHUNKS * CH) + j * CH)
    return carry

  lax.fori_loop(0, CHUNKS, body, 0)

  @pl.when(w < EXTRA)
  def _():
    do_chunk(FULL + w * CH)

  @pl.when(w == EXTRA)
  def _():
    pltpu.sync_copy(seg_hbm.at[pl.ds(TAIL_OFF, TAIL)], idx32)
    pltpu.sync_copy(feat_hbm.at[pl.ds(TAIL_OFF, TAIL)], rows.at[pl.ds(0, TAIL)])
    pltpu.sync_copy(rows.at[pl.ds(0, TAIL)], acc.at[idx32], add=True)
    count_ids(idx32, TAIL)

  plsc.subcore_barrier()

  pltpu.sync_copy(acc.at[sl], psums_hbm.at[c, sl])
  pltpu.sync_copy(cnt, pcnts_hbm.at[w])


_sc_pool = functools.partial(
    pl.kernel,
    out_type=(
        jax.ShapeDtypeStruct((NC, S, D), jnp.float32),
        jax.ShapeDtypeStruct((NW, S), jnp.float32),
    ),
    mesh=plsc.VectorSubcoreMesh(
        core_axis_name="c", subcore_axis_name="s",
        num_cores=NC, num_subcores=NS),
    scratch_types=[
        pltpu.VMEM_SHARED((S, D), jnp.float32),
        pltpu.VMEM((CH, D), jnp.float32),
        pltpu.VMEM((CH,), jnp.int32),
        pltpu.VMEM((TAIL,), jnp.int32),
        pltpu.VMEM((S,), jnp.float32),
        pltpu.VMEM((16, D), jnp.float32),
    ],
    compiler_params=pltpu.CompilerParams(needs_layout_passes=False),
)(_sc_body)


def _combine_body(ps_ref, pc_ref, o_ref):
  sums = ps_ref[0] + ps_ref[1]
  counts = jnp.sum(pc_ref[...], axis=0, keepdims=True)
  counts_col = jnp.transpose(counts)
  o_ref[...] = sums / jnp.clip(counts_col, 1.0, None)


def kernel(feat, segment_ids, num_graphs):
  seg = segment_ids.astype(jnp.int32)
  psums, pcnts = _sc_pool(feat, seg)
  out = pl.pallas_call(
      _combine_body,
      out_shape=jax.ShapeDtypeStruct((S, D), jnp.float32),
  )(psums, pcnts)
  return out


# speedup vs baseline: 8.6233x; 1.5423x over previous
import functools

import jax
import jax.numpy as jnp
from jax import lax
from jax.experimental import pallas as pl
from jax.experimental.pallas import tpu as pltpu
from jax.experimental.pallas import tpu_sc as plsc

N_ROWS = 100000
D = 128
S = 256
NC = 2
NS = 16
NW = NC * NS
CH = 128
FULL = (N_ROWS // (NW * CH)) * NW * CH
CHUNKS = FULL // (NW * CH)
EXTRA = (N_ROWS - FULL) // CH
TAIL = N_ROWS - FULL - EXTRA * CH
TAIL_OFF = FULL + EXTRA * CH


BIG = 384
NBIG = CHUNKS * CH // BIG


def _sc_body(feat_hbm, seg_hbm, psums_hbm, pcnts_hbm,
             acc, rows0, rows1, idx_all, idx_e, idx32, cnt, zbuf,
             sem_g0, sem_g1, sem_s, sem_i):
  c = lax.axis_index("c")
  s = lax.axis_index("s")
  w = s * NC + c
  base = w * (CHUNKS * CH)

  zv = jnp.zeros((16,), jnp.float32)
  ov = jnp.ones((16,), jnp.float32)
  for i in range(16):
    for k in range(D // 16):
      zbuf[i, pl.ds(k * 16, 16)] = zv
  for i in range(S // 16):
    cnt[pl.ds(i * 16, 16)] = zv

  sl = pl.ds(s * 16, 16)
  pltpu.sync_copy(zbuf, acc.at[sl])
  plsc.subcore_barrier()

  def count_ids(idx_vals):
    plsc.addupdate_scatter(cnt, [idx_vals], ov)

  bufs = (rows0, rows1)
  gsems = (sem_g0, sem_g1)

  idx_dmas = [
      pltpu.async_copy(seg_hbm.at[pl.ds(base + r * CH, CH)], idx_all.at[r],
                       sem_i)
      for r in range(CHUNKS)
  ]
  gathers = [pltpu.async_copy(feat_hbm.at[pl.ds(base, BIG)], bufs[0], sem_g0)]
  for d in idx_dmas:
    d.wait()

  pending_scatters = []
  for b in range(NBIG):
    for d in pending_scatters:
      d.wait()
    pending_scatters = []
    if b + 1 < NBIG:
      gathers.append(
          pltpu.async_copy(feat_hbm.at[pl.ds(base + (b + 1) * BIG, BIG)],
                           bufs[(b + 1) % 2], gsems[(b + 1) % 2]))
    gathers[b].wait()
    buf = bufs[b % 2]
    for k in range(BIG // CH):
      r = b * (BIG // CH) + k
      pending_scatters.append(
          pltpu.async_copy(buf.at[pl.ds(k * CH, CH)], acc.at[idx_all.at[r]],
                           sem_s, add=True))
    for k in range(BIG // CH):
      r = b * (BIG // CH) + k
      for q in range(CH // 16):
        count_ids(idx_all[r, pl.ds(q * 16, 16)])
  for d in pending_scatters:
    d.wait()

  @pl.when(w < EXTRA)
  def _():
    off = FULL + w * CH
    pltpu.sync_copy(seg_hbm.at[pl.ds(off, CH)], idx_e)
    pltpu.sync_copy(feat_hbm.at[pl.ds(off, CH)], rows0.at[pl.ds(0, CH)])
    pltpu.sync_copy(rows0.at[pl.ds(0, CH)], acc.at[idx_e], add=True)
    for q in range(CH // 16):
      count_ids(idx_e[pl.ds(q * 16, 16)])

  @pl.when(w == EXTRA)
  def _():
    pltpu.sync_copy(seg_hbm.at[pl.ds(TAIL_OFF, TAIL)], idx32)
    pltpu.sync_copy(feat_hbm.at[pl.ds(TAIL_OFF, TAIL)], rows1.at[pl.ds(0, TAIL)])
    pltpu.sync_copy(rows1.at[pl.ds(0, TAIL)], acc.at[idx32], add=True)
    for q in range(TAIL // 16):
      count_ids(idx32[pl.ds(q * 16, 16)])

  plsc.subcore_barrier()

  pltpu.sync_copy(acc.at[sl], psums_hbm.at[c, sl])
  pltpu.sync_copy(cnt, pcnts_hbm.at[w])


_sc_pool = functools.partial(
    pl.kernel,
    out_type=(
        jax.ShapeDtypeStruct((NC, S, D), jnp.float32),
        jax.ShapeDtypeStruct((NW, S), jnp.float32),
    ),
    mesh=plsc.VectorSubcoreMesh(
        core_axis_name="c", subcore_axis_name="s",
        num_cores=NC, num_subcores=NS),
    scratch_types=[
        pltpu.VMEM_SHARED((S, D), jnp.float32),
        pltpu.VMEM((BIG, D), jnp.float32),
        pltpu.VMEM((BIG, D), jnp.float32),
        pltpu.VMEM((CHUNKS, CH), jnp.int32),
        pltpu.VMEM((CH,), jnp.int32),
        pltpu.VMEM((TAIL,), jnp.int32),
        pltpu.VMEM((S,), jnp.float32),
        pltpu.VMEM((16, D), jnp.float32),
        pltpu.SemaphoreType.DMA,
        pltpu.SemaphoreType.DMA,
        pltpu.SemaphoreType.DMA,
        pltpu.SemaphoreType.DMA,
    ],
    compiler_params=pltpu.CompilerParams(needs_layout_passes=False),
)(_sc_body)


def _combine_body(ps_ref, pc_ref, o_ref):
  sums = ps_ref[0] + ps_ref[1]
  counts = jnp.sum(pc_ref[...], axis=0, keepdims=True)
  counts_col = jnp.transpose(counts)
  o_ref[...] = sums / jnp.clip(counts_col, 1.0, None)


def kernel(feat, segment_ids, num_graphs):
  seg = segment_ids.astype(jnp.int32)
  psums, pcnts = _sc_pool(feat, seg)
  out = pl.pallas_call(
      _combine_body,
      out_shape=jax.ShapeDtypeStruct((S, D), jnp.float32),
  )(psums, pcnts)
  return out
